# baseline (speedup 1.0000x reference)
import functools

import jax
import jax.numpy as jnp
from jax import lax
from jax.experimental import pallas as pl
from jax.experimental.pallas import tpu as pltpu
from jax.experimental.pallas import tpu_sc as plsc

B = 16384
D = 128
P = 10
K = 20
R = P + K
RPAD = 32
NC = 2
NS = 16
NW = NC * NS
BW = B // NW
C = 16
NCHUNK = BW // C
NG = 5
G = C * R // NG
SECT = 128
CPS = SECT // C
NPASS = 3
PP = R // NPASS


def _sc_dots(in_W, out_W, labels, comb):
  mesh = plsc.VectorSubcoreMesh(core_axis_name="c", subcore_axis_name="s")

  @functools.partial(
      pl.kernel,
      out_type=jax.ShapeDtypeStruct((B * RPAD,), jnp.float32),
      mesh=mesh,
      compiler_params=pltpu.CompilerParams(needs_layout_passes=False),
      scratch_types=[
          pltpu.VMEM((BW,), jnp.int32),
          pltpu.VMEM((BW * R,), jnp.int32),
          pltpu.VMEM((SECT, D), jnp.float32),
          pltpu.VMEM((C * R, D), jnp.float32),
          pltpu.VMEM((BW * RPAD,), jnp.float32),
          pltpu.SemaphoreType.DMA,
          pltpu.SemaphoreType.DMA,
          pltpu.SemaphoreType.DMA,
          pltpu.SemaphoreType.DMA,
          pltpu.SemaphoreType.DMA,
      ],
  )
  def k(in_hbm, out_hbm, lab_hbm, comb_hbm, dots_hbm,
        lab_v, comb_v, inbuf, rowbuf, dots_v, sem0, sem1, sem2, sem3, sem4):
    sems = (sem0, sem1, sem2, sem3, sem4)
    wid = lax.axis_index("s") * NC + lax.axis_index("c")
    base = wid * BW
    pltpu.sync_copy(lab_hbm.at[pl.ds(base, BW)], lab_v)
    pltpu.sync_copy(comb_hbm.at[pl.ds(base * R, BW * R)], comb_v)

    def qcopy(chunk, q):
      off = chunk * (C * R) + q * G
      return (out_hbm.at[comb_v.at[pl.ds(off, G)]],
              rowbuf.at[pl.ds(q * G, G)])

    lanes = lax.iota(jnp.int32, 16)

    def do_pass(chunk, sect, npass):
      zeros = jnp.zeros((16,), jnp.float32)
      inrow = jnp.full((16,), chunk * C - sect * SECT, jnp.int32) + lanes

      def dstep(i, accs):
        dvec = jnp.bitwise_and(jnp.full((16,), i, jnp.int32) + lanes, D - 1)
        xg = plsc.load_gather(inbuf, [inrow, dvec])
        out = []
        for p in range(npass * PP, (npass + 1) * PP):
          v = plsc.load_gather(rowbuf, [p * C + lanes, dvec])
          out.append(accs[p - npass * PP] + v * xg)
        return tuple(out)

      accs = lax.fori_loop(0, D, dstep, (zeros,) * PP, unroll=2)
      for p in range(npass * PP, (npass + 1) * PP):
        dots_v[pl.ds(chunk * (C * RPAD) + p * C, 16)] = accs[p - npass * PP]

    for sect in range(BW // SECT):
      insrc = in_hbm.at[lab_v.at[pl.ds(sect * SECT, SECT)]]
      pltpu.async_copy(insrc, inbuf, sem0)
      pltpu.make_async_copy(insrc, inbuf, sem0).wait()

      @pl.loop(sect * CPS, (sect + 1) * CPS)
      def chunk_body(chunk, sect=sect):
        for q in range(NG):
          src, dst = qcopy(chunk, q)
          pltpu.async_copy(src, dst, sems[q])
        for q in range(2):
          src, dst = qcopy(chunk, q)
          pltpu.make_async_copy(src, dst, sems[q]).wait()
        do_pass(chunk, sect, 0)
        for q in range(2, 4):
          src, dst = qcopy(chunk, q)
          pltpu.make_async_copy(src, dst, sems[q]).wait()
        do_pass(chunk, sect, 1)
        src, dst = qcopy(chunk, 4)
        pltpu.make_async_copy(src, dst, sems[4]).wait()
        do_pass(chunk, sect, 2)

    pltpu.sync_copy(dots_v, dots_hbm.at[pl.ds(base * RPAD, BW * RPAD)])

  return k(in_W, out_W, labels, comb)


def _tc_loss(dots):
  def body(dref, oref):
    x = dref[...]
    pos = lax.broadcasted_iota(jnp.int32, x.shape, 1)
    sign = jnp.where(pos < P, 1.0, -1.0)
    v = jax.nn.log_sigmoid(sign * x)
    v = jnp.where(pos < R, v, 0.0)
    oref[...] = -jnp.sum(v, axis=1)

  return pl.pallas_call(
      body,
      out_shape=jax.ShapeDtypeStruct((B // 16, 16), jnp.float32),
  )(dots)


def kernel(in_W, out_W, input_labels, pos_labels, neg_labels):
  labels = input_labels.astype(jnp.int32)
  comb = jnp.concatenate(
      [pos_labels.astype(jnp.int32), neg_labels.astype(jnp.int32)],
      axis=1).reshape(B // C, C, R).transpose(0, 2, 1).reshape(-1)
  dots = _sc_dots(in_W, out_W, labels, comb).reshape(B // 16, RPAD, 16)
  return _tc_loss(dots).reshape(B)
